# baseline (speedup 1.0000x reference)
import jax
import jax.numpy as jnp
from jax.experimental import pallas as pl
from jax.experimental.pallas import tpu as pltpu

_NUM_STEPS = 1000
_B, _L, _D = 64, 4096, 128
_TL = 512


def _make_sqrt_tables():
    betas = jnp.linspace(0.0001, 0.02, _NUM_STEPS)
    alphas_bar = jnp.cumprod(1.0 - betas)
    return jnp.sqrt(alphas_bar), jnp.sqrt(1.0 - alphas_bar)


def _blend_body(t_ref, sa_tab_ref, sb_tab_ref, x_ref, n_ref, o_ref):
    b = pl.program_id(0)
    tb = t_ref[b]
    sa = sa_tab_ref[tb]
    sb = sb_tab_ref[tb]
    o_ref[...] = sa * x_ref[...] + sb * n_ref[...]


def kernel(x, noise, t):
    t = t.astype(jnp.int32)
    sa_tab, sb_tab = _make_sqrt_tables()
    grid = (_B, _L // _TL)
    smem = pl.BlockSpec(memory_space=pltpu.SMEM)
    big = pl.BlockSpec((1, _TL, _D), lambda b, l: (b, l, 0))
    return pl.pallas_call(
        _blend_body,
        grid=grid,
        in_specs=[smem, smem, smem, big, big],
        out_specs=big,
        out_shape=jax.ShapeDtypeStruct((_B, _L, _D), jnp.float32),
    )(t, sa_tab, sb_tab, x, noise)


# speedup vs baseline: 2.6715x; 2.6715x over previous
import jax
import jax.numpy as jnp
from jax.experimental import pallas as pl
from jax.experimental.pallas import tpu as pltpu

_NUM_STEPS = 1000
_B, _L, _D = 64, 4096, 128
_TL = 4096


def _make_sqrt_tables():
    betas = jnp.linspace(0.0001, 0.02, _NUM_STEPS)
    alphas_bar = jnp.cumprod(1.0 - betas)
    return jnp.sqrt(alphas_bar), jnp.sqrt(1.0 - alphas_bar)


def _blend_body(t_ref, sa_tab_ref, sb_tab_ref, x_ref, n_ref, o_ref):
    b = pl.program_id(0)
    tb = t_ref[b]
    sa = sa_tab_ref[tb]
    sb = sb_tab_ref[tb]
    o_ref[...] = sa * x_ref[...] + sb * n_ref[...]


def kernel(x, noise, t):
    t = t.astype(jnp.int32)
    sa_tab, sb_tab = _make_sqrt_tables()
    grid = (_B, _L // _TL)
    smem = pl.BlockSpec(memory_space=pltpu.SMEM)
    big = pl.BlockSpec((1, _TL, _D), lambda b, l: (b, l, 0))
    return pl.pallas_call(
        _blend_body,
        grid=grid,
        in_specs=[smem, smem, smem, big, big],
        out_specs=big,
        out_shape=jax.ShapeDtypeStruct((_B, _L, _D), jnp.float32),
    )(t, sa_tab, sb_tab, x, noise)


# speedup vs baseline: 2.7624x; 1.0340x over previous
import jax
import jax.numpy as jnp
from jax.experimental import pallas as pl
from jax.experimental.pallas import tpu as pltpu

_NUM_STEPS = 1000
_B, _L, _D = 64, 4096, 128
_TL = 4096


def _make_sqrt_tables():
    betas = jnp.linspace(0.0001, 0.02, _NUM_STEPS)
    alphas_bar = jnp.cumprod(1.0 - betas)
    return jnp.sqrt(alphas_bar), jnp.sqrt(1.0 - alphas_bar)


_NB = 2


def _blend_body(t_ref, sa_tab_ref, sb_tab_ref, x_ref, n_ref, o_ref):
    b = pl.program_id(0)
    for i in range(_NB):
        tb = t_ref[b * _NB + i]
        sa = sa_tab_ref[tb]
        sb = sb_tab_ref[tb]
        o_ref[i] = sa * x_ref[i] + sb * n_ref[i]


def kernel(x, noise, t):
    t = t.astype(jnp.int32)
    sa_tab, sb_tab = _make_sqrt_tables()
    grid = (_B // _NB,)
    smem = pl.BlockSpec(memory_space=pltpu.SMEM)
    big = pl.BlockSpec((_NB, _TL, _D), lambda b: (b, 0, 0))
    return pl.pallas_call(
        _blend_body,
        grid=grid,
        in_specs=[smem, smem, smem, big, big],
        out_specs=big,
        out_shape=jax.ShapeDtypeStruct((_B, _L, _D), jnp.float32),
    )(t, sa_tab, sb_tab, x, noise)
